# Initial kernel scaffold; baseline (speedup 1.0000x reference)
import jax
import jax.numpy as jnp
from jax.experimental import pallas as pl


def kernel(emb_support, emb_query, glo_support, glo_query, thresh, memory_keys, memory_values):
    raise NotImplementedError("write your pallas kernel here")


# speedup vs baseline: 6.6001x; 6.6001x over previous
import jax
import jax.numpy as jnp
from jax.experimental import pallas as pl
from jax.experimental.pallas import tpu as pltpu

_T, _N, _D, _M = 2, 100, 128, 1024
_R = _T * _N
_MARGIN = 0.5


def _l2n(x):
    return x / jnp.maximum(jnp.sqrt(jnp.sum(x * x, axis=-1, keepdims=True)), 1e-12)


def _body(emb_ref, glo_ref, th_ref, k_ref, v_ref,
          nemb_ref, eg_ref, lk_ref, lv_ref, ls_ref):
    emb = emb_ref[...]
    glo = glo_ref[...]
    ne = _l2n(emb)
    ng = _l2n(glo)
    nemb_ref[...] = ne

    kmat = k_ref[...]
    vmat = v_ref[...]
    sim_kv = jax.lax.dot_general(ne, kmat, (((1,), (1,)), ((), ())),
                                 preferred_element_type=jnp.float32)
    sim_vk = jax.lax.dot_general(ng, vmat, (((1,), (1,)), ((), ())),
                                 preferred_element_type=jnp.float32)

    th0 = th_ref[0]
    th1 = th_ref[1]
    th2 = th_ref[2]
    th3 = th_ref[3]

    pos_mask = sim_kv > th0
    pos_score = jnp.where(pos_mask, sim_kv, 0.0)

    eg = ng + jax.lax.dot_general(pos_score, vmat, (((1,), (0,)), ((), ())),
                                  preferred_element_type=jnp.float32)
    eg_ref[...] = _l2n(eg)

    diff = sim_vk - sim_kv
    ls_ref[...] = jnp.sum(diff * diff, keepdims=True).reshape(1, 1) / (_R * _M)

    iota = jax.lax.broadcasted_iota(jnp.int32, (_R, _M), 1)
    big = jnp.int32(2 ** 30)

    def pick(masked, other, is_min):
        if is_min:
            ext = jnp.min(masked, axis=1, keepdims=True)
        else:
            ext = jnp.max(masked, axis=1, keepdims=True)
        cand = jnp.where(masked == ext, iota, big)
        idx = jnp.min(cand, axis=1, keepdims=True)
        val = jnp.sum(jnp.where(iota == idx, other, 0.0), axis=1, keepdims=True)
        return val

    inf = jnp.float32(jnp.inf)

    pos_any_v = jnp.max(pos_mask.astype(jnp.float32), axis=1, keepdims=True)
    neg_mask_v = sim_kv < th1
    neg_any_v = jnp.max(neg_mask_v.astype(jnp.float32), axis=1, keepdims=True)
    pv = pick(jnp.where(pos_mask, sim_kv, inf), sim_vk, True)
    nv = pick(jnp.where(neg_mask_v, sim_kv, -inf), sim_vk, False)
    mean_v = jnp.sum(pos_any_v * pv - neg_any_v * nv, keepdims=True).reshape(1, 1) / _R
    lv_ref[...] = jnp.maximum(-mean_v + _MARGIN, 0.0)

    pos_mask_k = sim_vk > th2
    neg_mask_k = sim_vk < th3
    pos_any_k = jnp.max(pos_mask_k.astype(jnp.float32), axis=1, keepdims=True)
    neg_any_k = jnp.max(neg_mask_k.astype(jnp.float32), axis=1, keepdims=True)
    pk = pick(jnp.where(pos_mask_k, sim_vk, inf), sim_kv, True)
    nk = pick(jnp.where(neg_mask_k, sim_vk, -inf), sim_kv, False)
    mean_k = jnp.sum(pos_any_k * pk - neg_any_k * nk, keepdims=True).reshape(1, 1) / _R
    lk_ref[...] = jnp.maximum(-mean_k + _MARGIN, 0.0)


def kernel(emb_support, emb_query, glo_support, glo_query, thresh,
           memory_keys, memory_values):
    emb = jnp.concatenate([emb_support, emb_query], axis=1).reshape(_R, _D)
    glo = jnp.concatenate([glo_support, glo_query], axis=1).reshape(_R, _D)

    out_shape = (
        jax.ShapeDtypeStruct((_R, _D), jnp.float32),
        jax.ShapeDtypeStruct((_R, _D), jnp.float32),
        jax.ShapeDtypeStruct((1, 1), jnp.float32),
        jax.ShapeDtypeStruct((1, 1), jnp.float32),
        jax.ShapeDtypeStruct((1, 1), jnp.float32),
    )
    in_specs = [
        pl.BlockSpec(memory_space=pltpu.VMEM),
        pl.BlockSpec(memory_space=pltpu.VMEM),
        pl.BlockSpec(memory_space=pltpu.SMEM),
        pl.BlockSpec(memory_space=pltpu.VMEM),
        pl.BlockSpec(memory_space=pltpu.VMEM),
    ]
    out_specs = (
        pl.BlockSpec(memory_space=pltpu.VMEM),
        pl.BlockSpec(memory_space=pltpu.VMEM),
        pl.BlockSpec(memory_space=pltpu.VMEM),
        pl.BlockSpec(memory_space=pltpu.VMEM),
        pl.BlockSpec(memory_space=pltpu.VMEM),
    )
    ne, eg, lk, lv, ls = pl.pallas_call(
        _body,
        out_shape=out_shape,
        in_specs=in_specs,
        out_specs=out_specs,
    )(emb, glo, thresh, memory_keys, memory_values)

    return (ne.reshape(_T, _N, _D), eg.reshape(_T, _N, _D),
            lk[0, 0], lv[0, 0], ls[0, 0])
